# baseline (speedup 1.0000x reference)
import functools

import jax
import jax.numpy as jnp
from jax import lax
from jax.experimental import pallas as pl
from jax.experimental.pallas import tpu as pltpu
from jax.experimental.pallas import tpu_sc as plsc

_B = 4096
_M = 4
_D = 64
_NGROUPS = 4096

_NC = 2
_NS = 16
_NW = _NC * _NS
_PW = _B // _NW
_GCH = 128


def _sc_gather(gi, ii, gm_flat, ut, it, gt):
    mesh = plsc.VectorSubcoreMesh(core_axis_name="c", subcore_axis_name="s")

    @functools.partial(
        pl.kernel,
        mesh=mesh,
        compiler_params=pltpu.CompilerParams(
            needs_layout_passes=False, use_tc_tiling_on_sc=False),
        out_type=(
            jax.ShapeDtypeStruct((_B * _M, _D), jnp.float32),
            jax.ShapeDtypeStruct((_B, _D), jnp.float32),
            jax.ShapeDtypeStruct((_B, _D), jnp.float32),
        ),
        scratch_types=[
            pltpu.VMEM((_PW,), jnp.int32),
            pltpu.VMEM((_PW,), jnp.int32),
            pltpu.VMEM((_NGROUPS * _M,), jnp.int32),
            pltpu.VMEM((_M, _GCH), jnp.int32),
            pltpu.VMEM((_PW * _M, _D), jnp.float32),
            pltpu.VMEM((_PW, _D), jnp.float32),
            pltpu.VMEM((_PW, _D), jnp.float32),
            pltpu.SemaphoreType.DMA,
        ],
    )
    def k(gi_hbm, ii_hbm, gm_hbm, ut_hbm, it_hbm, gt_hbm,
          me_out, ir_out, gr_out,
          gid_v, iid_v, gm_v, uidx_v, ur_v, ir_v, gr_v, sem):
        w = lax.axis_index("s") * _NC + lax.axis_index("c")
        base = w * _PW
        pltpu.sync_copy(gi_hbm.at[pl.ds(base, _PW)], gid_v)
        pltpu.sync_copy(ii_hbm.at[pl.ds(base, _PW)], iid_v)
        pltpu.sync_copy(gm_hbm, gm_v)

        lane = lax.iota(jnp.int32, 16)
        for i in range(_PW // 16):
            g16 = gid_v[pl.ds(i * 16, 16)]
            for m in range(_M):
                u16 = plsc.load_gather(gm_v, [g16 * _M + m])
                flat = (i * 16 + lane) * _M + m
                row = flat // _GCH
                col = flat % _GCH
                plsc.store_scatter(uidx_v, [row, col], u16)

        cps = []
        for j in range(_M):
            cps.append(pltpu.async_copy(
                ut_hbm.at[uidx_v.at[j]],
                ur_v.at[pl.ds(j * _GCH, _GCH)], sem))
        cps.append(pltpu.async_copy(it_hbm.at[iid_v], ir_v, sem))
        cps.append(pltpu.async_copy(gt_hbm.at[gid_v], gr_v, sem))
        for c in cps:
            c.wait()

        pltpu.sync_copy(ur_v, me_out.at[pl.ds(base * _M, _PW * _M)])
        pltpu.sync_copy(ir_v, ir_out.at[pl.ds(base, _PW)])
        pltpu.sync_copy(gr_v, gr_out.at[pl.ds(base, _PW)])

    return k(gi, ii, gm_flat, ut, it, gt)


_BLK = 512


def _tc_body(me_ref, it_ref, gr_ref, w1u_ref, w1i_ref, b1_ref, w2_ref, b2_ref,
             wc_ref, bc_ref, wp1_ref, bp1_ref, wp2_ref, bp2_ref,
             y_ref, aw_ref, ty_ref):
    me = me_ref[...]
    item = it_ref[...]
    grp = gr_ref[...]
    w1u = w1u_ref[...]
    b1 = b1_ref[...]
    w2 = w2_ref[...]

    t = jnp.dot(item, w1i_ref[...]) + b1
    cols = []
    for m in range(_M):
        mem = me[:, m * _D:(m + 1) * _D]
        h = jnp.maximum(jnp.dot(mem, w1u) + t, 0.0)
        cols.append(jnp.dot(h, w2))
    logits = jnp.concatenate(cols, axis=1) + b2_ref[...]

    mx = jnp.max(logits, axis=1, keepdims=True)
    e = jnp.exp(logits - mx)
    aw = e / jnp.sum(e, axis=1, keepdims=True)

    mw = jnp.max(aw, axis=1, keepdims=True)
    iota4 = lax.broadcasted_iota(jnp.int32, (_BLK, _M), 1).astype(jnp.float32)
    idx = jnp.min(jnp.where(aw >= mw, iota4, float(_M)), axis=1, keepdims=True)
    oh = (iota4 == idx).astype(jnp.float32)

    wc = wc_ref[...]
    bc = bc_ref[...]
    diff = aw * (wc[:, 1:2] - wc[:, 0:1]) + (bc[:, 1:2] - bc[:, 0:1])
    pred = (diff > 0.0).astype(jnp.float32)
    ptype = jnp.sum(oh * pred, axis=1, keepdims=True)

    wsel = jnp.where(ptype == 1.0, oh, aw)
    g = wsel[:, 0:1] * me[:, 0:_D]
    for m in range(1, _M):
        g = g + wsel[:, m:m + 1] * me[:, m * _D:(m + 1) * _D]

    ge = g + grp
    el = ge * item
    new = jnp.concatenate([el, ge, item], axis=1)
    p = jnp.maximum(jnp.dot(new, wp1_ref[...]) + bp1_ref[...], 0.0)
    y = jax.nn.sigmoid(jnp.dot(p, wp2_ref[...]) + bp2_ref[...])

    y_ref[...] = y
    aw_ref[...] = aw
    ty_ref[...] = ptype


def _tc_dense(me, item_rows, group_rows, w1u, w1i, b1, w2, b2, wc, bc,
              wp1, bp1, wp2, bp2):
    grid = _B // _BLK
    full = lambda a: pl.BlockSpec(a.shape, lambda i: (0,) * a.ndim)
    return pl.pallas_call(
        _tc_body,
        grid=(grid,),
        in_specs=[
            pl.BlockSpec((_BLK, _M * _D), lambda i: (i, 0)),
            pl.BlockSpec((_BLK, _D), lambda i: (i, 0)),
            pl.BlockSpec((_BLK, _D), lambda i: (i, 0)),
            full(w1u), full(w1i), full(b1), full(w2), full(b2),
            full(wc), full(bc), full(wp1), full(bp1), full(wp2), full(bp2),
        ],
        out_specs=[
            pl.BlockSpec((_BLK, 1), lambda i: (i, 0)),
            pl.BlockSpec((_BLK, _M), lambda i: (i, 0)),
            pl.BlockSpec((_BLK, 1), lambda i: (i, 0)),
        ],
        out_shape=[
            jax.ShapeDtypeStruct((_B, 1), jnp.float32),
            jax.ShapeDtypeStruct((_B, _M), jnp.float32),
            jax.ShapeDtypeStruct((_B, 1), jnp.float32),
        ],
    )(me, item_rows, group_rows, w1u, w1i, b1, w2, b2, wc, bc,
      wp1, bp1, wp2, bp2)


def kernel(group_inputs, item_inputs, group_members, user_table, item_table,
           group_table, W1, b1, W2, b2, Wc, bc, Wp1, bp1, Wp2, bp2):
    me, item_rows, group_rows = _sc_gather(
        group_inputs, item_inputs, group_members.reshape(-1),
        user_table, item_table, group_table)
    me = me.reshape(_B, _M * _D)

    y, aw, ty = _tc_dense(
        me, item_rows, group_rows,
        W1[:_D], W1[_D:], b1.reshape(1, 16), W2, b2.reshape(1, 1),
        Wc, bc.reshape(1, 2), Wp1, bp1.reshape(1, 8), Wp2, bp2.reshape(1, 1))
    return y, aw, ty.reshape(_B)
